# baseline (speedup 1.0000x reference)
import functools

import jax
import jax.numpy as jnp
from jax import lax
from jax.experimental import pallas as pl
from jax.experimental.pallas import tpu as pltpu
from jax.experimental.pallas import tpu_sc as plsc

NUM_NODES = 1024
DIM = 64
BLOCK_M = 64


def _scores_body(w_ref, c_ref, s_ref):
    w = w_ref[...]
    c = c_ref[0]
    e = jnp.sum(w * c[None, :, None], axis=1)
    m = jnp.max(e, axis=-1, keepdims=True)
    p = jnp.exp(e - m)
    s_ref[...] = p / jnp.sum(p, axis=-1, keepdims=True)


def _compute_scores(Wt, c2):
    N = NUM_NODES
    return pl.pallas_call(
        _scores_body,
        grid=(N // BLOCK_M,),
        in_specs=[
            pl.BlockSpec((BLOCK_M, DIM, N), lambda i: (i, 0, 0)),
            pl.BlockSpec((1, DIM), lambda i: (0, 0)),
        ],
        out_specs=pl.BlockSpec((BLOCK_M, N), lambda i: (i, 0)),
        out_shape=jax.ShapeDtypeStruct((N, N), jnp.float32),
    )(Wt, c2)


def _make_sc_gather(B, D):
    info = plsc.get_sparse_core_info()
    NC, NS = info.num_cores, info.num_subcores
    NW = NC * NS
    b_per_w = B // NW
    mesh = plsc.VectorSubcoreMesh(core_axis_name="c", subcore_axis_name="s")

    @functools.partial(
        pl.kernel,
        mesh=mesh,
        out_type=jax.ShapeDtypeStruct((B, D), jnp.float32),
        scratch_types=[
            pltpu.VMEM((b_per_w,), jnp.int32),
            pltpu.VMEM((b_per_w, D), jnp.float32),
            pltpu.SemaphoreType.DMA,
            pltpu.SemaphoreType.DMA,
        ],
    )
    def gather_k(table_hbm, idx_hbm, out_hbm, idx_v, rows_v, gsem, wsem):
        wid = lax.axis_index("s") * NC + lax.axis_index("c")
        base = wid * b_per_w
        half = b_per_w // 2
        pltpu.sync_copy(idx_hbm.at[pl.ds(base, b_per_w)], idx_v)
        g0 = pltpu.async_copy(table_hbm.at[idx_v.at[pl.ds(0, half)]],
                              rows_v.at[pl.ds(0, half)], gsem)
        g1 = pltpu.async_copy(table_hbm.at[idx_v.at[pl.ds(half, half)]],
                              rows_v.at[pl.ds(half, half)], gsem)
        g0.wait()
        w0 = pltpu.async_copy(rows_v.at[pl.ds(0, half)],
                              out_hbm.at[pl.ds(base, half)], wsem)
        g1.wait()
        w1 = pltpu.async_copy(rows_v.at[pl.ds(half, half)],
                              out_hbm.at[pl.ds(base + half, half)], wsem)
        w0.wait()
        w1.wait()

    return gather_k


def kernel(node_indices, context_vector, W):
    Wt = W.transpose(0, 2, 1)
    c2 = context_vector.reshape(1, DIM)
    scores = _compute_scores(Wt, c2)
    gather = _make_sc_gather(node_indices.shape[0], NUM_NODES)
    return gather(scores, node_indices.astype(jnp.int32))


# speedup vs baseline: 1.0061x; 1.0061x over previous
import functools

import jax
import jax.numpy as jnp
from jax import lax
from jax.experimental import pallas as pl
from jax.experimental.pallas import tpu as pltpu
from jax.experimental.pallas import tpu_sc as plsc

NUM_NODES = 1024
DIM = 64
BLOCK_M = 64


def _scores_body(w_ref, c_ref, s_ref):
    w = w_ref[...]
    c = c_ref[0]
    e = jnp.sum(w * c[None, :, None], axis=1)
    m = jnp.max(e, axis=-1, keepdims=True)
    p = jnp.exp(e - m)
    s_ref[...] = p / jnp.sum(p, axis=-1, keepdims=True)


def _compute_scores(Wt, c2):
    N = NUM_NODES
    return pl.pallas_call(
        _scores_body,
        grid=(N // BLOCK_M,),
        in_specs=[
            pl.BlockSpec((BLOCK_M, DIM, N), lambda i: (i, 0, 0)),
            pl.BlockSpec((1, DIM), lambda i: (0, 0)),
        ],
        out_specs=pl.BlockSpec((BLOCK_M, N), lambda i: (i, 0)),
        out_shape=jax.ShapeDtypeStruct((N, N), jnp.float32),
    )(Wt, c2)


def _make_sc_gather(B, D):
    info = plsc.get_sparse_core_info()
    NC, NS = info.num_cores, info.num_subcores
    NW = NC * NS
    b_per_w = B // NW
    mesh = plsc.VectorSubcoreMesh(core_axis_name="c", subcore_axis_name="s")

    @functools.partial(
        pl.kernel,
        mesh=mesh,
        out_type=jax.ShapeDtypeStruct((B, D), jnp.float32),
        scratch_types=[
            pltpu.VMEM((b_per_w,), jnp.int32),
            pltpu.VMEM((b_per_w, D), jnp.float32),
            pltpu.SemaphoreType.DMA,
        ],
    )
    def gather_k(table_hbm, idx_hbm, out_hbm, idx_v, rows_v, sem):
        wid = lax.axis_index("s") * NC + lax.axis_index("c")
        base = wid * b_per_w
        pltpu.sync_copy(idx_hbm.at[pl.ds(base, b_per_w)], idx_v)
        pltpu.async_copy(table_hbm.at[idx_v], rows_v, sem).wait()
        pltpu.sync_copy(rows_v, out_hbm.at[pl.ds(base, b_per_w)])

    return gather_k


def kernel(node_indices, context_vector, W):
    Wt = W.transpose(0, 2, 1)
    c2 = context_vector.reshape(1, DIM)
    scores = _compute_scores(Wt, c2)
    gather = _make_sc_gather(node_indices.shape[0], NUM_NODES)
    return gather(scores, node_indices.astype(jnp.int32))
